# Initial kernel scaffold; baseline (speedup 1.0000x reference)
import functools

import jax
import jax.numpy as jnp
from jax import lax
from jax.experimental import pallas as pl
from jax.experimental.pallas import tpu as pltpu
from jax.experimental.pallas import tpu_sc as plsc

N = 10000
NP = 10240
E = 320000
LATENT = 128
MAT = 16
OUT = 128

NC = 2
NS = 16
NW = NC * NS
EPW = E // NW
K = 80
CHUNKS = EPW // K
ROWS_PER_TILE = NP // NS

_mesh = plsc.VectorSubcoreMesh(core_axis_name="c", subcore_axis_name="s")


@functools.partial(
    pl.kernel,
    out_type=jax.ShapeDtypeStruct((NC, NP, 16), jnp.float32),
    mesh=_mesh,
    scratch_types=[
        pltpu.VMEM((CHUNKS, K), jnp.int32),
        pltpu.VMEM((K, 16), jnp.float32),
        pltpu.VMEM_SHARED((NP, 16), jnp.float32),
    ],
)
def _deg_kernel(dst_hbm, ones_hbm, zeros_hbm, out_hbm, idx_v, ones_v, acc_sh):
    cid = lax.axis_index("c")
    sid = lax.axis_index("s")
    wid = cid * NS + sid
    pltpu.sync_copy(dst_hbm.at[wid], idx_v)
    pltpu.sync_copy(ones_hbm, ones_v)
    pltpu.sync_copy(
        zeros_hbm.at[pl.ds(sid * ROWS_PER_TILE, ROWS_PER_TILE)],
        acc_sh.at[pl.ds(sid * ROWS_PER_TILE, ROWS_PER_TILE)],
    )
    plsc.subcore_barrier()

    def body(c, carry):
        pltpu.sync_copy(ones_v, acc_sh.at[idx_v.at[c]], add=True)
        return carry

    lax.fori_loop(0, CHUNKS, body, 0)
    plsc.subcore_barrier()
    pltpu.sync_copy(
        acc_sh.at[pl.ds(sid * ROWS_PER_TILE, ROWS_PER_TILE)],
        out_hbm.at[cid, pl.ds(sid * ROWS_PER_TILE, ROWS_PER_TILE)],
    )


@functools.partial(
    pl.kernel,
    out_type=jax.ShapeDtypeStruct((NC, NP, LATENT), jnp.float32),
    mesh=_mesh,
    scratch_types=[
        pltpu.VMEM((CHUNKS, K), jnp.int32),
        pltpu.VMEM((CHUNKS, K), jnp.int32),
        pltpu.VMEM((K, LATENT), jnp.float32),
        pltpu.VMEM_SHARED((NP, LATENT), jnp.float32),
        pltpu.SemaphoreType.DMA,
    ],
)
def _edge_kernel(hs_hbm, src_hbm, dst_hbm, zeros_hbm, out_hbm,
                 src_v, dst_v, rows_v, acc_sh, sem):
    cid = lax.axis_index("c")
    sid = lax.axis_index("s")
    wid = cid * NS + sid
    pltpu.sync_copy(src_hbm.at[wid], src_v)
    pltpu.sync_copy(dst_hbm.at[wid], dst_v)
    pltpu.sync_copy(
        zeros_hbm.at[pl.ds(sid * ROWS_PER_TILE, ROWS_PER_TILE)],
        acc_sh.at[pl.ds(sid * ROWS_PER_TILE, ROWS_PER_TILE)],
    )
    plsc.subcore_barrier()

    def body(c, carry):
        pltpu.async_copy(hs_hbm.at[src_v.at[c]], rows_v, sem).wait()
        pltpu.sync_copy(rows_v, acc_sh.at[dst_v.at[c]], add=True)
        return carry

    lax.fori_loop(0, CHUNKS, body, 0)
    plsc.subcore_barrier()
    pltpu.sync_copy(
        acc_sh.at[pl.ds(sid * ROWS_PER_TILE, ROWS_PER_TILE)],
        out_hbm.at[cid, pl.ds(sid * ROWS_PER_TILE, ROWS_PER_TILE)],
    )


RB = 1024
GRID = NP // RB


def _dis(da_ref, db_ref):
    deg = 1.0 + da_ref[:, 0:1] + db_ref[:, 0:1]
    return lax.rsqrt(deg)


def _l1_body(z_ref, mp_ref, da_ref, db_ref, w1z_ref, w1m_ref, o_ref):
    h = jnp.dot(z_ref[...], w1z_ref[...], preferred_element_type=jnp.float32)
    h = h + jnp.dot(mp_ref[...], w1m_ref[...],
                    preferred_element_type=jnp.float32)
    o_ref[...] = h * _dis(da_ref, db_ref)


def _l2_body(aa_ref, ab_ref, hs_ref, da_ref, db_ref, b1_ref, w2_ref, o_ref):
    dis = _dis(da_ref, db_ref)
    x = dis * (aa_ref[...] + ab_ref[...] + hs_ref[...]) + b1_ref[...]
    x = jnp.maximum(x, 0.0)
    o_ref[...] = jnp.dot(x, w2_ref[...],
                         preferred_element_type=jnp.float32) * dis


def _fin_body(aa_ref, ab_ref, hs_ref, da_ref, db_ref, b2_ref, o_ref):
    dis = _dis(da_ref, db_ref)
    o_ref[...] = dis * (aa_ref[...] + ab_ref[...] + hs_ref[...]) + b2_ref[...]


def _row_spec(width):
    return pl.BlockSpec((RB, width), lambda i: (i, 0))


def _full_spec(shape):
    return pl.BlockSpec(shape, lambda i: (0,) * len(shape))


_l1_call = pl.pallas_call(
    _l1_body,
    out_shape=jax.ShapeDtypeStruct((NP, LATENT), jnp.float32),
    grid=(GRID,),
    in_specs=[
        _row_spec(LATENT), _row_spec(MAT), _row_spec(16), _row_spec(16),
        _full_spec((LATENT, LATENT)), _full_spec((MAT, LATENT)),
    ],
    out_specs=_row_spec(LATENT),
)

_l2_call = pl.pallas_call(
    _l2_body,
    out_shape=jax.ShapeDtypeStruct((NP, OUT), jnp.float32),
    grid=(GRID,),
    in_specs=[
        _row_spec(LATENT), _row_spec(LATENT), _row_spec(LATENT),
        _row_spec(16), _row_spec(16),
        _full_spec((1, LATENT)), _full_spec((LATENT, OUT)),
    ],
    out_specs=_row_spec(OUT),
)

_fin_call = pl.pallas_call(
    _fin_body,
    out_shape=jax.ShapeDtypeStruct((NP, OUT), jnp.float32),
    grid=(GRID,),
    in_specs=[
        _row_spec(OUT), _row_spec(OUT), _row_spec(OUT),
        _row_spec(16), _row_spec(16),
        _full_spec((1, OUT)),
    ],
    out_specs=_row_spec(OUT),
)


def kernel(z, edge_index, material_params, W1, b1, W2, b2):
    src = edge_index[0].astype(jnp.int32).reshape(NW, CHUNKS, K)
    dst = edge_index[1].astype(jnp.int32).reshape(NW, CHUNKS, K)
    zp = jnp.pad(z, ((0, NP - N), (0, 0)))
    mpp = jnp.pad(material_params, ((0, NP - N), (0, 0)))
    zeros16 = jnp.zeros((NP, 16), jnp.float32)
    zeros128 = jnp.zeros((NP, LATENT), jnp.float32)
    ones16 = jnp.ones((K, 16), jnp.float32)

    deg = _deg_kernel(dst, ones16, zeros16)
    da, db = deg[0], deg[1]

    hs1 = _l1_call(zp, mpp, da, db, W1[:LATENT], W1[LATENT:])
    acc1 = _edge_kernel(hs1, src, dst, zeros128)
    hs2 = _l2_call(acc1[0], acc1[1], hs1, da, db, b1.reshape(1, LATENT), W2)
    acc2 = _edge_kernel(hs2, src, dst, zeros128)
    out = _fin_call(acc2[0], acc2[1], hs2, da, db, b2.reshape(1, OUT))
    return out[:N]


# speedup vs baseline: 16.0130x; 16.0130x over previous
import functools

import jax
import jax.numpy as jnp
from jax import lax
from jax.experimental import pallas as pl
from jax.experimental.pallas import tpu as pltpu
from jax.experimental.pallas import tpu_sc as plsc

N = 10000
NP = 10240
E = 320000
LATENT = 128
MAT = 16
OUT = 128

NC = 2
NS = 16
NW = NC * NS
EPW = E // NW
K = 80
CHUNKS = EPW // K
ROWS_PER_TILE = NP // NS

_mesh = plsc.VectorSubcoreMesh(core_axis_name="c", subcore_axis_name="s",
                               num_cores=NC, num_subcores=NS)


@functools.partial(
    pl.kernel,
    out_type=jax.ShapeDtypeStruct((NC, NP, LATENT), jnp.float32),
    mesh=_mesh,
    scratch_types=[
        pltpu.VMEM((CHUNKS, K), jnp.int32),
        pltpu.VMEM((K, LATENT), jnp.float32),
        pltpu.VMEM_SHARED((NP, LATENT), jnp.float32),
    ],
)
def _deg_kernel(dst_hbm, ones_hbm, zeros_hbm, out_hbm, idx_v, ones_v, acc_sh):
    cid = lax.axis_index("c")
    sid = lax.axis_index("s")
    wid = cid * NS + sid
    pltpu.sync_copy(dst_hbm.at[wid], idx_v)
    pltpu.sync_copy(ones_hbm, ones_v)
    pltpu.sync_copy(
        zeros_hbm.at[pl.ds(sid * ROWS_PER_TILE, ROWS_PER_TILE)],
        acc_sh.at[pl.ds(sid * ROWS_PER_TILE, ROWS_PER_TILE)],
    )
    plsc.subcore_barrier()

    def body(c, carry):
        pltpu.sync_copy(ones_v, acc_sh.at[idx_v.at[c]], add=True)
        return carry

    lax.fori_loop(0, CHUNKS, body, 0)
    plsc.subcore_barrier()
    pltpu.sync_copy(
        acc_sh.at[pl.ds(sid * ROWS_PER_TILE, ROWS_PER_TILE)],
        out_hbm.at[cid, pl.ds(sid * ROWS_PER_TILE, ROWS_PER_TILE)],
    )


@functools.partial(
    pl.kernel,
    out_type=jax.ShapeDtypeStruct((NC, NP, LATENT), jnp.float32),
    mesh=_mesh,
    scratch_types=[
        pltpu.VMEM((CHUNKS, K), jnp.int32),
        pltpu.VMEM((CHUNKS, K), jnp.int32),
        pltpu.VMEM((K, LATENT), jnp.float32),
        pltpu.VMEM_SHARED((NP, LATENT), jnp.float32),
        pltpu.SemaphoreType.DMA,
    ],
)
def _edge_kernel(hs_hbm, src_hbm, dst_hbm, zeros_hbm, out_hbm,
                 src_v, dst_v, rows_v, acc_sh, sem):
    cid = lax.axis_index("c")
    sid = lax.axis_index("s")
    wid = cid * NS + sid
    pltpu.sync_copy(src_hbm.at[wid], src_v)
    pltpu.sync_copy(dst_hbm.at[wid], dst_v)
    pltpu.sync_copy(
        zeros_hbm.at[pl.ds(sid * ROWS_PER_TILE, ROWS_PER_TILE)],
        acc_sh.at[pl.ds(sid * ROWS_PER_TILE, ROWS_PER_TILE)],
    )
    plsc.subcore_barrier()

    def body(c, carry):
        pltpu.async_copy(hs_hbm.at[src_v.at[c]], rows_v, sem).wait()
        pltpu.sync_copy(rows_v, acc_sh.at[dst_v.at[c]], add=True)
        return carry

    lax.fori_loop(0, CHUNKS, body, 0)
    plsc.subcore_barrier()
    pltpu.sync_copy(
        acc_sh.at[pl.ds(sid * ROWS_PER_TILE, ROWS_PER_TILE)],
        out_hbm.at[cid, pl.ds(sid * ROWS_PER_TILE, ROWS_PER_TILE)],
    )


RB = 1024
GRID = NP // RB


def _dis(da_ref, db_ref):
    deg = 1.0 + da_ref[:, 0:1] + db_ref[:, 0:1]
    return lax.rsqrt(deg)


def _l1_body(z_ref, mp_ref, da_ref, db_ref, w1z_ref, w1m_ref, o_ref):
    h = jnp.dot(z_ref[...], w1z_ref[...], preferred_element_type=jnp.float32)
    h = h + jnp.dot(mp_ref[...], w1m_ref[...],
                    preferred_element_type=jnp.float32)
    o_ref[...] = h * _dis(da_ref, db_ref)


def _l2_body(aa_ref, ab_ref, hs_ref, da_ref, db_ref, b1_ref, w2_ref, o_ref):
    dis = _dis(da_ref, db_ref)
    x = dis * (aa_ref[...] + ab_ref[...] + hs_ref[...]) + b1_ref[...]
    x = jnp.maximum(x, 0.0)
    o_ref[...] = jnp.dot(x, w2_ref[...],
                         preferred_element_type=jnp.float32) * dis


def _fin_body(aa_ref, ab_ref, hs_ref, da_ref, db_ref, b2_ref, o_ref):
    dis = _dis(da_ref, db_ref)
    o_ref[...] = dis * (aa_ref[...] + ab_ref[...] + hs_ref[...]) + b2_ref[...]


def _row_spec(width):
    return pl.BlockSpec((RB, width), lambda i: (i, 0))


def _full_spec(shape):
    return pl.BlockSpec(shape, lambda i: (0,) * len(shape))


_l1_call = pl.pallas_call(
    _l1_body,
    out_shape=jax.ShapeDtypeStruct((NP, LATENT), jnp.float32),
    grid=(GRID,),
    in_specs=[
        _row_spec(LATENT), _row_spec(MAT), _row_spec(LATENT), _row_spec(LATENT),
        _full_spec((LATENT, LATENT)), _full_spec((MAT, LATENT)),
    ],
    out_specs=_row_spec(LATENT),
)

_l2_call = pl.pallas_call(
    _l2_body,
    out_shape=jax.ShapeDtypeStruct((NP, OUT), jnp.float32),
    grid=(GRID,),
    in_specs=[
        _row_spec(LATENT), _row_spec(LATENT), _row_spec(LATENT),
        _row_spec(LATENT), _row_spec(LATENT),
        _full_spec((1, LATENT)), _full_spec((LATENT, OUT)),
    ],
    out_specs=_row_spec(OUT),
)

_fin_call = pl.pallas_call(
    _fin_body,
    out_shape=jax.ShapeDtypeStruct((NP, OUT), jnp.float32),
    grid=(GRID,),
    in_specs=[
        _row_spec(OUT), _row_spec(OUT), _row_spec(OUT),
        _row_spec(LATENT), _row_spec(LATENT),
        _full_spec((1, OUT)),
    ],
    out_specs=_row_spec(OUT),
)


def kernel(z, edge_index, material_params, W1, b1, W2, b2):
    src = edge_index[0].astype(jnp.int32).reshape(NW, CHUNKS, K)
    dst = edge_index[1].astype(jnp.int32).reshape(NW, CHUNKS, K)
    zp = jnp.pad(z, ((0, NP - N), (0, 0)))
    mpp = jnp.pad(material_params, ((0, NP - N), (0, 0)))
    zeros128 = jnp.zeros((NP, LATENT), jnp.float32)
    ones128 = jnp.ones((K, LATENT), jnp.float32)

    deg = _deg_kernel(dst, ones128, zeros128)
    da, db = deg[0], deg[1]

    hs1 = _l1_call(zp, mpp, da, db, W1[:LATENT], W1[LATENT:])
    acc1 = _edge_kernel(hs1, src, dst, zeros128)
    hs2 = _l2_call(acc1[0], acc1[1], hs1, da, db, b1.reshape(1, LATENT), W2)
    acc2 = _edge_kernel(hs2, src, dst, zeros128)
    out = _fin_call(acc2[0], acc2[1], hs2, da, db, b2.reshape(1, OUT))
    return out[:N]
